# baseline (speedup 1.0000x reference)
import functools

import jax
import jax.numpy as jnp
from jax.experimental import pallas as pl
from jax.experimental.pallas import tpu as pltpu

_NEG = -9e15
_ALPHA = 0.2


def _gat_kernel(nf_ref, adj_ref, w_ref, b_ref, asrc_ref, adst_ref, out_ref,
                nfa_ref, nfa16_ref, s_ref, t_ref, tm_ref, fb_ref,
                *, num_heads, c_head, blk):
    i = pl.program_id(0)
    band = 2 * c_head

    @pl.when(i == 0)
    def _init():
        nfa = jax.lax.dot_general(
            nf_ref[...], w_ref[...],
            (((1,), (1,)), ((), ())),
            preferred_element_type=jnp.float32) + b_ref[...]
        nfa_ref[...] = nfa
        nfa16_ref[...] = nfa.astype(jnp.bfloat16)
        hb = num_heads * band
        row = jax.lax.broadcasted_iota(jnp.int32, (hb, num_heads), 0) // band
        col = jax.lax.broadcasted_iota(jnp.int32, (hb, num_heads), 1)
        mask = (row == col).astype(jnp.float32)
        a_src = asrc_ref[...] * mask
        a_dst = adst_ref[...] * mask
        s_ref[...] = jnp.dot(nfa, a_src, preferred_element_type=jnp.float32)
        t = jax.lax.dot_general(
            a_dst, nfa, (((0,), (1,)), ((), ())),
            preferred_element_type=jnp.float32)
        t_ref[...] = t
        tm_ref[...] = jnp.max(t, axis=1, keepdims=True)
        n = nfa.shape[0]
        fb_ref[...] = jax.lax.dot_general(
            jnp.full((1, n), 1.0 / n, dtype=jnp.float32), nfa,
            (((1,), (0,)), ((), ())),
            preferred_element_type=jnp.float32)

    madd = jnp.where(adj_ref[...] != 0, 0.0, _NEG)
    for h in range(num_heads):
        s_h = s_ref[pl.ds(i * blk, blk), h:h + 1]
        t_h = t_ref[h:h + 1, :]
        mb = s_h + tm_ref[h:h + 1, 0:1]
        mb = jnp.maximum(mb, _ALPHA * mb)
        x = s_h + t_h
        masked = jnp.maximum(x, _ALPHA * x) + madd
        e = jnp.exp(masked - mb).astype(jnp.bfloat16)
        agg = jnp.dot(e, nfa16_ref[:, h * band:h * band + c_head + 1],
                      preferred_element_type=jnp.float32)
        den = agg[:, c_head:c_head + 1]
        out_h = agg[:, :c_head] * (1.0 / den)
        out_ref[:, h * c_head:(h + 1) * c_head] = jnp.where(
            den > 0.0, out_h, fb_ref[0:1, h * band:h * band + c_head])


def kernel(node_feats, adj_matrix, W, b, a):
    B, N, c_in = node_feats.shape
    num_heads = a.shape[0]
    c_head = a.shape[1] // 2
    hc = num_heads * c_head
    band = 2 * c_head
    hb = num_heads * band

    nf = node_feats.reshape(N, c_in)
    adj = adj_matrix.reshape(N, N)
    w_pad = jnp.pad(W.reshape(num_heads, c_head, c_in),
                    ((0, 0), (0, band - c_head), (0, 0))).reshape(hb, c_in)
    b_pad = jnp.pad(b.reshape(num_heads, c_head),
                    ((0, 0), (0, band - c_head)))
    b_pad = b_pad.at[:, c_head].set(1.0).reshape(1, hb)
    a_src = jnp.pad(a[:, :c_head],
                    ((0, 0), (0, band - c_head))).reshape(hb, 1)
    a_dst = jnp.pad(a[:, c_head:],
                    ((0, 0), (0, band - c_head))).reshape(hb, 1)

    blk = 256
    out = pl.pallas_call(
        functools.partial(_gat_kernel, num_heads=num_heads, c_head=c_head,
                          blk=blk),
        grid=(N // blk,),
        in_specs=[
            pl.BlockSpec((N, c_in), lambda i: (0, 0)),
            pl.BlockSpec((blk, N), lambda i: (i, 0)),
            pl.BlockSpec((hb, c_in), lambda i: (0, 0)),
            pl.BlockSpec((1, hb), lambda i: (0, 0)),
            pl.BlockSpec((hb, 1), lambda i: (0, 0)),
            pl.BlockSpec((hb, 1), lambda i: (0, 0)),
        ],
        out_specs=pl.BlockSpec((blk, hc), lambda i: (i, 0)),
        out_shape=jax.ShapeDtypeStruct((N, hc), jnp.float32),
        scratch_shapes=[
            pltpu.VMEM((N, hb), jnp.float32),
            pltpu.VMEM((N, hb), jnp.bfloat16),
            pltpu.VMEM((N, num_heads), jnp.float32),
            pltpu.VMEM((num_heads, N), jnp.float32),
            pltpu.VMEM((num_heads, 1), jnp.float32),
            pltpu.VMEM((1, hb), jnp.float32),
        ],
    )(nf, adj, w_pad, b_pad, a_src, a_dst)
    return out.reshape(B, N, hc)


# speedup vs baseline: 1.0584x; 1.0584x over previous
import functools

import jax
import jax.numpy as jnp
from jax.experimental import pallas as pl
from jax.experimental.pallas import tpu as pltpu

_NEG = -9e15
_ALPHA = 0.2


def _gat_kernel(nf_ref, adj_ref, w_ref, b_ref, asrc_ref, adst_ref, out_ref,
                nfa_ref, nfa16_ref, s_ref, t_ref, tm_ref, fb_ref,
                *, num_heads, c_head, blk):
    i = pl.program_id(0)
    band = 2 * c_head

    @pl.when(i == 0)
    def _init():
        nfa = jax.lax.dot_general(
            nf_ref[...], w_ref[...],
            (((1,), (1,)), ((), ())),
            preferred_element_type=jnp.float32) + b_ref[...]
        nfa_ref[...] = nfa
        nfa16_ref[...] = nfa.astype(jnp.bfloat16)
        hb = num_heads * band
        row = jax.lax.broadcasted_iota(jnp.int32, (hb, num_heads), 0) // band
        col = jax.lax.broadcasted_iota(jnp.int32, (hb, num_heads), 1)
        mask = (row == col).astype(jnp.float32)
        a_src = asrc_ref[...] * mask
        a_dst = adst_ref[...] * mask
        s_ref[...] = jnp.dot(nfa, a_src, preferred_element_type=jnp.float32)
        t = jax.lax.dot_general(
            a_dst, nfa, (((0,), (1,)), ((), ())),
            preferred_element_type=jnp.float32)
        t_ref[...] = t
        tm_ref[...] = jnp.max(t, axis=1, keepdims=True)
        n = nfa.shape[0]
        fb_ref[...] = jax.lax.dot_general(
            jnp.full((1, n), 1.0 / n, dtype=jnp.float32), nfa,
            (((1,), (0,)), ((), ())),
            preferred_element_type=jnp.float32)

    madd = jnp.where(adj_ref[...] != 0, 0.0, _NEG)
    for h in range(num_heads):
        s_h = s_ref[pl.ds(i * blk, blk), h:h + 1]
        t_h = t_ref[h:h + 1, :]
        mb = s_h + tm_ref[h:h + 1, 0:1]
        mb = jnp.maximum(mb, _ALPHA * mb)
        x = s_h + t_h
        masked = jnp.maximum(x, _ALPHA * x) + madd
        e = jnp.exp(masked - mb).astype(jnp.bfloat16)
        agg = jnp.dot(e, nfa16_ref[...],
                      preferred_element_type=jnp.float32)
        den = agg[:, h * band + c_head:h * band + c_head + 1]
        out_h = agg[:, h * band:h * band + c_head] * (1.0 / den)
        out_ref[:, h * c_head:(h + 1) * c_head] = jnp.where(
            den > 0.0, out_h, fb_ref[0:1, h * band:h * band + c_head])


def kernel(node_feats, adj_matrix, W, b, a):
    B, N, c_in = node_feats.shape
    num_heads = a.shape[0]
    c_head = a.shape[1] // 2
    hc = num_heads * c_head
    band = 2 * c_head
    hb = num_heads * band

    nf = node_feats.reshape(N, c_in)
    adj = adj_matrix.reshape(N, N)
    w_pad = jnp.pad(W.reshape(num_heads, c_head, c_in),
                    ((0, 0), (0, band - c_head), (0, 0))).reshape(hb, c_in)
    b_pad = jnp.pad(b.reshape(num_heads, c_head),
                    ((0, 0), (0, band - c_head)))
    b_pad = b_pad.at[:, c_head].set(1.0).reshape(1, hb)
    a_src = jnp.pad(a[:, :c_head],
                    ((0, 0), (0, band - c_head))).reshape(hb, 1)
    a_dst = jnp.pad(a[:, c_head:],
                    ((0, 0), (0, band - c_head))).reshape(hb, 1)

    blk = 256
    out = pl.pallas_call(
        functools.partial(_gat_kernel, num_heads=num_heads, c_head=c_head,
                          blk=blk),
        grid=(N // blk,),
        in_specs=[
            pl.BlockSpec((N, c_in), lambda i: (0, 0)),
            pl.BlockSpec((blk, N), lambda i: (i, 0)),
            pl.BlockSpec((hb, c_in), lambda i: (0, 0)),
            pl.BlockSpec((1, hb), lambda i: (0, 0)),
            pl.BlockSpec((hb, 1), lambda i: (0, 0)),
            pl.BlockSpec((hb, 1), lambda i: (0, 0)),
        ],
        out_specs=pl.BlockSpec((blk, hc), lambda i: (i, 0)),
        out_shape=jax.ShapeDtypeStruct((N, hc), jnp.float32),
        scratch_shapes=[
            pltpu.VMEM((N, hb), jnp.float32),
            pltpu.VMEM((N, hb), jnp.bfloat16),
            pltpu.VMEM((N, num_heads), jnp.float32),
            pltpu.VMEM((num_heads, N), jnp.float32),
            pltpu.VMEM((num_heads, 1), jnp.float32),
            pltpu.VMEM((1, hb), jnp.float32),
        ],
    )(nf, adj, w_pad, b_pad, a_src, a_dst)
    return out.reshape(B, N, hc)
